# Initial kernel scaffold; baseline (speedup 1.0000x reference)
import functools

import jax
import jax.numpy as jnp
from jax import lax
from jax.experimental import pallas as pl
from jax.experimental.pallas import tpu as pltpu
from jax.experimental.pallas import tpu_sc as plsc

N = 10000
E = 320000
D = 128
NC = 2
NS = 16
NW = NC * NS
K = 80
C = E // (NW * K)
RP = N // NS
WD = 16

_mesh = plsc.VectorSubcoreMesh(core_axis_name="c", subcore_axis_name="s")


@functools.partial(
    pl.kernel,
    out_type=jax.ShapeDtypeStruct((NC, N, WD), jnp.float32),
    mesh=_mesh,
    scratch_types=[
        pltpu.VMEM_SHARED((N, WD), jnp.float32),
        pltpu.VMEM((C, K), jnp.int32),
        pltpu.VMEM((K, WD), jnp.float32),
    ],
)
def _deg_sc(dst_hbm, zeros_hbm, ones_hbm, out_hbm, acc, didx, ones_v):
    c = lax.axis_index("c")
    s = lax.axis_index("s")
    wid = s * NC + c
    pltpu.sync_copy(zeros_hbm.at[pl.ds(s * RP, RP)], acc.at[pl.ds(s * RP, RP)])
    pltpu.sync_copy(ones_hbm, ones_v)
    pltpu.sync_copy(dst_hbm.at[pl.ds(wid * C, C)], didx)
    plsc.subcore_barrier()

    def body(j, carry):
        pltpu.sync_copy(ones_v, acc.at[didx.at[j]], add=True)
        return carry

    lax.fori_loop(0, C, body, 0)
    plsc.subcore_barrier()
    pltpu.sync_copy(acc.at[pl.ds(s * RP, RP)], out_hbm.at[c, pl.ds(s * RP, RP)])


@functools.partial(
    pl.kernel,
    out_type=jax.ShapeDtypeStruct((NC, N, D), jnp.float32),
    mesh=_mesh,
    scratch_types=[
        pltpu.VMEM_SHARED((N, D), jnp.float32),
        pltpu.VMEM((C, K), jnp.int32),
        pltpu.VMEM((C, K), jnp.int32),
        pltpu.VMEM((K, D), jnp.float32),
        pltpu.SemaphoreType.DMA,
    ],
)
def _agg_sc(y_hbm, src_hbm, dst_hbm, zeros_hbm, out_hbm, acc, sidx, didx, rows, sem):
    c = lax.axis_index("c")
    s = lax.axis_index("s")
    wid = s * NC + c
    pltpu.sync_copy(zeros_hbm.at[pl.ds(s * RP, RP)], acc.at[pl.ds(s * RP, RP)])
    pltpu.sync_copy(src_hbm.at[pl.ds(wid * C, C)], sidx)
    pltpu.sync_copy(dst_hbm.at[pl.ds(wid * C, C)], didx)
    plsc.subcore_barrier()

    def body(j, carry):
        pltpu.async_copy(y_hbm.at[sidx.at[j]], rows, sem).wait()
        pltpu.sync_copy(rows, acc.at[didx.at[j]], add=True)
        return carry

    lax.fori_loop(0, C, body, 0)
    plsc.subcore_barrier()
    pltpu.sync_copy(acc.at[pl.ds(s * RP, RP)], out_hbm.at[c, pl.ds(s * RP, RP)])


BN = 1000


def _tc1_body(x_ref, w_ref, degp_ref, dinv_ref, y_ref):
    deg = 1.0 + degp_ref[0, :, :1] + degp_ref[1, :, :1]
    dinv = lax.rsqrt(deg)
    dinv_ref[...] = jnp.broadcast_to(dinv, (BN, WD))
    xw = jnp.dot(x_ref[...], w_ref[...], precision=lax.Precision.HIGHEST,
                 preferred_element_type=jnp.float32)
    y_ref[...] = xw * dinv


_tc1 = pl.pallas_call(
    _tc1_body,
    grid=(N // BN,),
    in_specs=[
        pl.BlockSpec((BN, D), lambda i: (i, 0)),
        pl.BlockSpec((D, D), lambda i: (0, 0)),
        pl.BlockSpec((NC, BN, WD), lambda i: (0, i, 0)),
    ],
    out_specs=[
        pl.BlockSpec((BN, WD), lambda i: (i, 0)),
        pl.BlockSpec((BN, D), lambda i: (i, 0)),
    ],
    out_shape=[
        jax.ShapeDtypeStruct((N, WD), jnp.float32),
        jax.ShapeDtypeStruct((N, D), jnp.float32),
    ],
)


def _tc2_body(aggp_ref, y1_ref, dinv_ref, b1_ref, w2_ref, y2_ref):
    dinv = dinv_ref[:, :1]
    pre = (aggp_ref[0] + aggp_ref[1] + y1_ref[...]) * dinv + b1_ref[...]
    h = jnp.where(pre > 0, pre, jnp.expm1(pre))
    hw = jnp.dot(h, w2_ref[...], precision=lax.Precision.HIGHEST,
                 preferred_element_type=jnp.float32)
    y2_ref[...] = hw * dinv


_tc2 = pl.pallas_call(
    _tc2_body,
    grid=(N // BN,),
    in_specs=[
        pl.BlockSpec((NC, BN, D), lambda i: (0, i, 0)),
        pl.BlockSpec((BN, D), lambda i: (i, 0)),
        pl.BlockSpec((BN, WD), lambda i: (i, 0)),
        pl.BlockSpec((1, D), lambda i: (0, 0)),
        pl.BlockSpec((D, D), lambda i: (0, 0)),
    ],
    out_specs=pl.BlockSpec((BN, D), lambda i: (i, 0)),
    out_shape=jax.ShapeDtypeStruct((N, D), jnp.float32),
)


def _tc3_body(aggp_ref, y2_ref, dinv_ref, b2_ref, out_ref):
    dinv = dinv_ref[:, :1]
    out_ref[...] = (aggp_ref[0] + aggp_ref[1] + y2_ref[...]) * dinv + b2_ref[...]


_tc3 = pl.pallas_call(
    _tc3_body,
    grid=(N // BN,),
    in_specs=[
        pl.BlockSpec((NC, BN, D), lambda i: (0, i, 0)),
        pl.BlockSpec((BN, D), lambda i: (i, 0)),
        pl.BlockSpec((BN, WD), lambda i: (i, 0)),
        pl.BlockSpec((1, D), lambda i: (0, 0)),
    ],
    out_specs=pl.BlockSpec((BN, D), lambda i: (i, 0)),
    out_shape=jax.ShapeDtypeStruct((N, D), jnp.float32),
)


def kernel(x, edge_index, W1, b1, W2, b2):
    src = edge_index[0].astype(jnp.int32).reshape(NW * C, K)
    dst = edge_index[1].astype(jnp.int32).reshape(NW * C, K)
    zeros_nd = jnp.zeros((N, D), jnp.float32)
    zeros_nw = jnp.zeros((N, WD), jnp.float32)
    ones_kw = jnp.ones((K, WD), jnp.float32)
    b1r = b1.reshape(1, D)
    b2r = b2.reshape(1, D)

    degp = _deg_sc(dst, zeros_nw, ones_kw)
    dinv, y1 = _tc1(x, W1, degp)
    agg1 = _agg_sc(y1, src, dst, zeros_nd)
    y2 = _tc2(agg1, y1, dinv, b1r, W2)
    agg2 = _agg_sc(y2, src, dst, zeros_nd)
    return _tc3(agg2, y2, dinv, b2r)


# speedup vs baseline: 18.9420x; 18.9420x over previous
import functools

import jax
import jax.numpy as jnp
from jax import lax
from jax.experimental import pallas as pl
from jax.experimental.pallas import tpu as pltpu
from jax.experimental.pallas import tpu_sc as plsc

N = 10000
E = 320000
D = 128
NC = 2
NS = 16
NW = NC * NS
K = 80
C = E // (NW * K)
WD = 16
NP = 10240
RPP = NP // NS

_mesh = plsc.VectorSubcoreMesh(core_axis_name="c", subcore_axis_name="s")


@functools.partial(
    pl.kernel,
    out_type=jax.ShapeDtypeStruct((NC, NP, WD), jnp.float32),
    mesh=_mesh,
    scratch_types=[
        pltpu.VMEM_SHARED((NP, WD), jnp.float32),
        pltpu.VMEM((C, K), jnp.int32),
        pltpu.VMEM((K, WD), jnp.float32),
    ],
)
def _deg_sc(dst_hbm, zeros_hbm, ones_hbm, out_hbm, acc, didx, ones_v):
    c = lax.axis_index("c")
    s = lax.axis_index("s")
    wid = s * NC + c
    pltpu.sync_copy(zeros_hbm.at[pl.ds(s * RPP, RPP)], acc.at[pl.ds(s * RPP, RPP)])
    pltpu.sync_copy(ones_hbm, ones_v)
    pltpu.sync_copy(dst_hbm.at[wid], didx)
    plsc.subcore_barrier()

    def body(j, carry):
        pltpu.sync_copy(ones_v, acc.at[didx.at[j]], add=True)
        return carry

    lax.fori_loop(0, C, body, 0)
    plsc.subcore_barrier()
    pltpu.sync_copy(acc.at[pl.ds(s * RPP, RPP)], out_hbm.at[c, pl.ds(s * RPP, RPP)])


@functools.partial(
    pl.kernel,
    out_type=jax.ShapeDtypeStruct((NC, NP, D), jnp.float32),
    mesh=_mesh,
    scratch_types=[
        pltpu.VMEM_SHARED((NP, D), jnp.float32),
        pltpu.VMEM((C, K), jnp.int32),
        pltpu.VMEM((C, K), jnp.int32),
        pltpu.VMEM((K, D), jnp.float32),
        pltpu.SemaphoreType.DMA,
    ],
)
def _agg_sc(y_hbm, src_hbm, dst_hbm, zeros_hbm, out_hbm, acc, sidx, didx, rows, sem):
    c = lax.axis_index("c")
    s = lax.axis_index("s")
    wid = s * NC + c
    pltpu.sync_copy(zeros_hbm.at[pl.ds(s * RPP, RPP)], acc.at[pl.ds(s * RPP, RPP)])
    pltpu.sync_copy(src_hbm.at[wid], sidx)
    pltpu.sync_copy(dst_hbm.at[wid], didx)
    plsc.subcore_barrier()

    def body(j, carry):
        pltpu.async_copy(y_hbm.at[sidx.at[j]], rows, sem).wait()
        pltpu.sync_copy(rows, acc.at[didx.at[j]], add=True)
        return carry

    lax.fori_loop(0, C, body, 0)
    plsc.subcore_barrier()
    pltpu.sync_copy(acc.at[pl.ds(s * RPP, RPP)], out_hbm.at[c, pl.ds(s * RPP, RPP)])


BN = 1000


def _tc1_body(x_ref, w_ref, degp_ref, dinv_ref, y_ref):
    deg = 1.0 + degp_ref[0, :, :1] + degp_ref[1, :, :1]
    dinv = lax.rsqrt(deg)
    dinv_ref[...] = jnp.broadcast_to(dinv, (BN, WD))
    xw = jnp.dot(x_ref[...], w_ref[...], precision=lax.Precision.HIGHEST,
                 preferred_element_type=jnp.float32)
    y_ref[...] = xw * dinv


_tc1 = pl.pallas_call(
    _tc1_body,
    grid=(N // BN,),
    in_specs=[
        pl.BlockSpec((BN, D), lambda i: (i, 0)),
        pl.BlockSpec((D, D), lambda i: (0, 0)),
        pl.BlockSpec((NC, BN, WD), lambda i: (0, i, 0)),
    ],
    out_specs=[
        pl.BlockSpec((BN, WD), lambda i: (i, 0)),
        pl.BlockSpec((BN, D), lambda i: (i, 0)),
    ],
    out_shape=[
        jax.ShapeDtypeStruct((N, WD), jnp.float32),
        jax.ShapeDtypeStruct((N, D), jnp.float32),
    ],
)


def _tc2_body(aggp_ref, y1_ref, dinv_ref, b1_ref, w2_ref, y2_ref):
    dinv = dinv_ref[:, :1]
    pre = (aggp_ref[0] + aggp_ref[1] + y1_ref[...]) * dinv + b1_ref[...]
    h = jnp.where(pre > 0, pre, jnp.exp(pre) - 1.0)
    hw = jnp.dot(h, w2_ref[...], precision=lax.Precision.HIGHEST,
                 preferred_element_type=jnp.float32)
    y2_ref[...] = hw * dinv


_tc2 = pl.pallas_call(
    _tc2_body,
    grid=(N // BN,),
    in_specs=[
        pl.BlockSpec((NC, BN, D), lambda i: (0, i, 0)),
        pl.BlockSpec((BN, D), lambda i: (i, 0)),
        pl.BlockSpec((BN, WD), lambda i: (i, 0)),
        pl.BlockSpec((1, D), lambda i: (0, 0)),
        pl.BlockSpec((D, D), lambda i: (0, 0)),
    ],
    out_specs=pl.BlockSpec((BN, D), lambda i: (i, 0)),
    out_shape=jax.ShapeDtypeStruct((N, D), jnp.float32),
)


def _tc3_body(aggp_ref, y2_ref, dinv_ref, b2_ref, out_ref):
    dinv = dinv_ref[:, :1]
    out_ref[...] = (aggp_ref[0] + aggp_ref[1] + y2_ref[...]) * dinv + b2_ref[...]


_tc3 = pl.pallas_call(
    _tc3_body,
    grid=(N // BN,),
    in_specs=[
        pl.BlockSpec((NC, BN, D), lambda i: (0, i, 0)),
        pl.BlockSpec((BN, D), lambda i: (i, 0)),
        pl.BlockSpec((BN, WD), lambda i: (i, 0)),
        pl.BlockSpec((1, D), lambda i: (0, 0)),
    ],
    out_specs=pl.BlockSpec((BN, D), lambda i: (i, 0)),
    out_shape=jax.ShapeDtypeStruct((N, D), jnp.float32),
)


def kernel(x, edge_index, W1, b1, W2, b2):
    src = edge_index[0].astype(jnp.int32).reshape(NW, C, K)
    dst = edge_index[1].astype(jnp.int32).reshape(NW, C, K)
    zeros_nd = jnp.zeros((NP, D), jnp.float32)
    zeros_nw = jnp.zeros((NP, WD), jnp.float32)
    ones_kw = jnp.ones((K, WD), jnp.float32)
    b1r = b1.reshape(1, D)
    b2r = b2.reshape(1, D)

    degp = _deg_sc(dst, zeros_nw, ones_kw)
    dinv, y1 = _tc1(x, W1, degp)
    agg1 = _agg_sc(y1, src, dst, zeros_nd)
    y2 = _tc2(agg1, y1, dinv, b1r, W2)
    agg2 = _agg_sc(y2, src, dst, zeros_nd)
    return _tc3(agg2, y2, dinv, b2r)
